# Initial kernel scaffold; baseline (speedup 1.0000x reference)
import functools

import jax
import jax.numpy as jnp
from jax import lax
from jax.experimental import pallas as pl
from jax.experimental.pallas import tpu as pltpu
from jax.experimental.pallas import tpu_sc as plsc

B = 8
N = 2048
KNN = 20
M = B * N
EPS = 1e-5
NEG = float("-inf")

BLK_R = 256
NC, NS = 2, 16
NW = NC * NS
NPTS = M // NW
GRP = 4
GK = GRP * KNN
NGRP = NPTS // GRP

BLK_M = 256


def _leaky(v):
    return jnp.where(v >= 0, v, 0.2 * v)


def _knn_proj_body(act, sr_ref, tr_ref, sc_ref, tc_ref, h_ref, ht_ref,
                   wa_ref, wd_ref, idx_ref, ya_ref, yb_ref, hat_ref, na_ref):
    b = pl.program_id(0)
    r = pl.program_id(1)

    @pl.when(r == 0)
    def _():
        hT = ht_ref[0]
        if act:
            hT = _leaky(hT * sc_ref[...] + tc_ref[...])
        hat_ref[...] = hT
        na_ref[...] = jnp.sum(hT * hT, axis=0, keepdims=True)

    hb = h_ref[0, pl.ds(r * BLK_R, BLK_R), :]
    if act:
        hb = _leaky(hb * sr_ref[...] + tr_ref[...])

    ya_ref[0] = jnp.dot(hb, wa_ref[...], preferred_element_type=jnp.float32)
    yb_ref[0] = jnp.dot(hb, wd_ref[...], preferred_element_type=jnp.float32)

    g = jnp.dot(hb, hat_ref[...], preferred_element_type=jnp.float32)
    nb = jnp.sum(hb * hb, axis=1, keepdims=True)
    p = 2.0 * g - nb - na_ref[...]

    iota = lax.broadcasted_iota(jnp.int32, (BLK_R, N), 1)
    cols = []
    for _ in range(KNN):
        m = jnp.max(p, axis=1, keepdims=True)
        a = jnp.min(jnp.where(p == m, iota, N), axis=1, keepdims=True)
        cols.append(a)
        p = jnp.where(iota == a, NEG, p)
    idx_ref[0] = jnp.concatenate(cols, axis=1) + b * N


def _knn_proj(h, ht, sr, tr, sc, tc, wa_t, wd_t, act):
    c = h.shape[2]
    f = wa_t.shape[1]
    const = lambda b, r: (0, 0)
    return pl.pallas_call(
        functools.partial(_knn_proj_body, act),
        grid=(B, N // BLK_R),
        in_specs=[
            pl.BlockSpec((1, c), const),
            pl.BlockSpec((1, c), const),
            pl.BlockSpec((c, 1), const),
            pl.BlockSpec((c, 1), const),
            pl.BlockSpec((1, N, c), lambda b, r: (b, 0, 0)),
            pl.BlockSpec((1, c, N), lambda b, r: (b, 0, 0)),
            pl.BlockSpec((c, f), const),
            pl.BlockSpec((c, f), const),
        ],
        out_specs=[
            pl.BlockSpec((1, BLK_R, KNN), lambda b, r: (b, r, 0)),
            pl.BlockSpec((1, BLK_R, f), lambda b, r: (b, r, 0)),
            pl.BlockSpec((1, BLK_R, f), lambda b, r: (b, r, 0)),
        ],
        out_shape=[
            jax.ShapeDtypeStruct((B, N, KNN), jnp.int32),
            jax.ShapeDtypeStruct((B, N, f), jnp.float32),
            jax.ShapeDtypeStruct((B, N, f), jnp.float32),
        ],
        scratch_shapes=[
            pltpu.VMEM((c, N), jnp.float32),
            pltpu.VMEM((1, N), jnp.float32),
        ],
    )(sr, tr, sc, tc, h, ht, wa_t, wd_t)


def _gather_reduce_body(f, idx_hbm, ya_hbm, yb_hbm, r_hbm, p1_hbm, p2_hbm,
                        idxg, rows, ybv, rv, acc1, acc2, sem):
    nch = f // 16
    wid = lax.axis_index("s") * NC + lax.axis_index("c")
    base_pt = wid * NPTS
    base_ix = wid * (NPTS * KNN)

    def zero_body(ci, carry):
        sl = pl.ds(ci * 16, 16)
        z = jnp.zeros((16,), jnp.float32)
        acc1[sl] = z
        acc2[sl] = z
        return carry

    lax.fori_loop(0, nch, zero_body, 0)

    def grp_body(g, carry):
        pltpu.sync_copy(idx_hbm.at[pl.ds(base_ix + g * GK, GK)], idxg)
        pltpu.async_copy(ya_hbm.at[idxg], rows, sem).wait()
        pltpu.sync_copy(yb_hbm.at[pl.ds(base_pt + g * GRP, GRP)], ybv)

        def ch_body(ci, inner):
            sl = pl.ds(ci * 16, 16)
            for pp in range(GRP):
                v = rows[pp * KNN, sl]
                mx = v
                sm = v
                sq = v * v
                for kk in range(1, KNN):
                    v = rows[pp * KNN + kk, sl]
                    mx = jnp.maximum(mx, v)
                    sm = sm + v
                    sq = sq + v * v
                yv = ybv[pp, sl]
                rv[pp, sl] = yv + mx
                acc1[sl] = acc1[sl] + sm + float(KNN) * yv
                acc2[sl] = acc2[sl] + sq + 2.0 * yv * sm + float(KNN) * yv * yv
            return inner

        lax.fori_loop(0, nch, ch_body, 0)
        pltpu.sync_copy(rv, r_hbm.at[pl.ds(base_pt + g * GRP, GRP)])
        return carry

    lax.fori_loop(0, NGRP, grp_body, 0)
    pltpu.sync_copy(acc1, p1_hbm.at[wid])
    pltpu.sync_copy(acc2, p2_hbm.at[wid])


def _gather_reduce(f, idx_flat, ya, yb):
    mesh = plsc.VectorSubcoreMesh(core_axis_name="c", subcore_axis_name="s",
                                  num_cores=NC, num_subcores=NS)
    fn = pl.kernel(
        functools.partial(_gather_reduce_body, f),
        out_type=[
            jax.ShapeDtypeStruct((M, f), jnp.float32),
            jax.ShapeDtypeStruct((NW, f), jnp.float32),
            jax.ShapeDtypeStruct((NW, f), jnp.float32),
        ],
        mesh=mesh,
        scratch_types=[
            pltpu.VMEM((GK,), jnp.int32),
            pltpu.VMEM((GK, f), jnp.float32),
            pltpu.VMEM((GRP, f), jnp.float32),
            pltpu.VMEM((GRP, f), jnp.float32),
            pltpu.VMEM((f,), jnp.float32),
            pltpu.VMEM((f,), jnp.float32),
            pltpu.SemaphoreType.DMA,
        ],
    )
    return fn(idx_flat, ya, yb)


def _final_body(r0, r1, r2, r3, s0, t0, s1, t1, s2, t2, s3, t3,
                wf_ref, y_ref, sum_ref, sq_ref):
    i = pl.program_id(0)
    rs = [r0, r1, r2, r3]
    ss = [s0, s1, s2, s3]
    ts = [t0, t1, t2, t3]
    parts = [
        _leaky(rs[j][...] * ss[j][...] + ts[j][...]) for j in range(4)
    ]
    hcat = jnp.concatenate(parts, axis=1)
    y = jnp.dot(hcat, wf_ref[...], preferred_element_type=jnp.float32)
    y_ref[...] = y

    @pl.when(i == 0)
    def _():
        sum_ref[...] = jnp.zeros_like(sum_ref)
        sq_ref[...] = jnp.zeros_like(sq_ref)

    sum_ref[...] += jnp.sum(y, axis=0, keepdims=True)
    sq_ref[...] += jnp.sum(y * y, axis=0, keepdims=True)


def _final_conv(rs, sts, wf_t, fdims, out_dim):
    const = lambda i: (0, 0)
    in_specs = [pl.BlockSpec((BLK_M, fdims[j]), lambda i: (i, 0))
                for j in range(4)]
    for j in range(4):
        in_specs.append(pl.BlockSpec((1, fdims[j]), const))
        in_specs.append(pl.BlockSpec((1, fdims[j]), const))
    in_specs.append(pl.BlockSpec((sum(fdims), out_dim), const))
    args = list(rs)
    for (s, t) in sts:
        args.append(s)
        args.append(t)
    args.append(wf_t)
    return pl.pallas_call(
        _final_body,
        grid=(M // BLK_M,),
        in_specs=in_specs,
        out_specs=[
            pl.BlockSpec((BLK_M, out_dim), lambda i: (i, 0)),
            pl.BlockSpec((1, out_dim), const),
            pl.BlockSpec((1, out_dim), const),
        ],
        out_shape=[
            jax.ShapeDtypeStruct((M, out_dim), jnp.float32),
            jax.ShapeDtypeStruct((1, out_dim), jnp.float32),
            jax.ShapeDtypeStruct((1, out_dim), jnp.float32),
        ],
    )(*args)


def _norm_body(s_ref, t_ref, y_ref, o_ref):
    o_ref[...] = _leaky(y_ref[...] * s_ref[...] + t_ref[...])


def _normalize(y, s, t, out_dim):
    const = lambda i: (0, 0)
    return pl.pallas_call(
        _norm_body,
        grid=(M // BLK_M,),
        in_specs=[
            pl.BlockSpec((1, out_dim), const),
            pl.BlockSpec((1, out_dim), const),
            pl.BlockSpec((BLK_M, out_dim), lambda i: (i, 0)),
        ],
        out_specs=pl.BlockSpec((BLK_M, out_dim), lambda i: (i, 0)),
        out_shape=jax.ShapeDtypeStruct((M, out_dim), jnp.float32),
    )(s, t, y)


def kernel(x, W0, g0, b0, W1, g1, b1, W2, g2, b2, W3, g3, b3, Wf, gf, bf):
    params = [(W0, g0, b0), (W1, g1, b1), (W2, g2, b2), (W3, g3, b3)]
    cnt = float(M * KNN)

    h = jnp.pad(x, ((0, 0), (0, 0), (0, 5)))
    scale = jnp.ones((8,), jnp.float32)
    shift = jnp.zeros((8,), jnp.float32)
    act = False

    feats = []
    for (W, g, bb) in params:
        f, c2 = W.shape
        c = c2 // 2
        wa = W[:, :c]
        wd = W[:, c:] - wa
        if not act:
            wa = jnp.pad(wa, ((0, 0), (0, 5)))
            wd = jnp.pad(wd, ((0, 0), (0, 5)))
            c = 8
        ht = jnp.transpose(h, (0, 2, 1))
        idx, ya, yb = _knn_proj(
            h, ht,
            scale.reshape(1, c), shift.reshape(1, c),
            scale.reshape(c, 1), shift.reshape(c, 1),
            wa.T, wd.T, act)
        r, p1, p2 = _gather_reduce(
            f, idx.reshape(M * KNN), ya.reshape(M, f), yb.reshape(M, f))
        s1 = jnp.sum(p1, axis=0)
        s2 = jnp.sum(p2, axis=0)
        mean = s1 / cnt
        var = s2 / cnt - mean * mean
        scale = g / jnp.sqrt(var + EPS)
        shift = bb - mean * scale
        feats.append((r, scale, shift))
        h = r.reshape(B, N, f)
        act = True

    fdims = [ft[0].shape[1] for ft in feats]
    out_dim = Wf.shape[0]
    y, ssum, ssq = _final_conv(
        [ft[0] for ft in feats],
        [(ft[1].reshape(1, -1), ft[2].reshape(1, -1)) for ft in feats],
        Wf.T, fdims, out_dim)
    mean = ssum[0] / float(M)
    var = ssq[0] / float(M) - mean * mean
    fscale = gf / jnp.sqrt(var + EPS)
    fshift = bf - mean * fscale
    out = _normalize(y, fscale.reshape(1, out_dim), fshift.reshape(1, out_dim),
                     out_dim)
    return out.reshape(B, N, out_dim)


# speedup vs baseline: 8.0611x; 8.0611x over previous
import functools

import jax
import jax.numpy as jnp
from jax import lax
from jax.experimental import pallas as pl
from jax.experimental.pallas import tpu as pltpu
from jax.experimental.pallas import tpu_sc as plsc

B = 8
N = 2048
KNN = 20
M = B * N
EPS = 1e-5
NEG = float("-inf")

BLK_R = 256
NC, NS = 2, 16
NW = NC * NS
NPTS = M // NW
GRP = 8
GK = GRP * KNN
NGRP = NPTS // GRP

BLK_C = 128
BLK_M = 256


def _leaky(v):
    return jnp.where(v >= 0, v, 0.2 * v)


def _knn_body(h_ref, ht_ref, xx_ref, xxt_ref, idx_ref, hat_ref):
    b = pl.program_id(0)
    r = pl.program_id(1)

    @pl.when(r == 0)
    def _():
        hat_ref[...] = ht_ref[0].astype(jnp.bfloat16)

    hb = h_ref[0, pl.ds(r * BLK_R, BLK_R), :].astype(jnp.bfloat16)
    g = jnp.dot(hb, hat_ref[...], preferred_element_type=jnp.float32)
    p = (-xx_ref[0]) - (-2.0 * g)
    p = p - xxt_ref[0]

    iota = lax.broadcasted_iota(jnp.int32, (BLK_R, N), 1)
    cols = []
    for _ in range(KNN):
        m = jnp.max(p, axis=1, keepdims=True)
        a = jnp.min(jnp.where(p == m, iota, N), axis=1, keepdims=True)
        cols.append(a)
        p = jnp.where(iota == a, NEG, p)
    idx_ref[0] = jnp.concatenate(cols, axis=1) + b * N


def _knn(h, ht, xx, xxt):
    c = h.shape[2]
    return pl.pallas_call(
        _knn_body,
        grid=(B, N // BLK_R),
        in_specs=[
            pl.BlockSpec((1, N, c), lambda b, r: (b, 0, 0)),
            pl.BlockSpec((1, c, N), lambda b, r: (b, 0, 0)),
            pl.BlockSpec((1, 1, N), lambda b, r: (b, 0, 0)),
            pl.BlockSpec((1, BLK_R, 1), lambda b, r: (b, r, 0)),
        ],
        out_specs=pl.BlockSpec((1, BLK_R, KNN), lambda b, r: (b, r, 0)),
        out_shape=jax.ShapeDtypeStruct((B, N, KNN), jnp.int32),
        scratch_shapes=[pltpu.VMEM((c, N), jnp.bfloat16)],
    )(h, ht, xx, xxt)


def _gather_body(idx_hbm, tab_hbm, out_hbm, idxg1, idxg2, rows1, rows2,
                 sem1, sem2):
    wid = lax.axis_index("s") * NC + lax.axis_index("c")
    base_ix = wid * (NPTS * KNN)

    def grp_body(g, carry):
        off = base_ix + g * GK
        pltpu.sync_copy(idx_hbm.at[pl.ds(off, GK // 2)], idxg1)
        pltpu.sync_copy(idx_hbm.at[pl.ds(off + GK // 2, GK // 2)], idxg2)
        cp1 = pltpu.async_copy(tab_hbm.at[idxg1], rows1, sem1)
        cp2 = pltpu.async_copy(tab_hbm.at[idxg2], rows2, sem2)
        cp1.wait()
        cp2.wait()
        pltpu.sync_copy(rows1, out_hbm.at[pl.ds(off, GK // 2)])
        pltpu.sync_copy(rows2, out_hbm.at[pl.ds(off + GK // 2, GK // 2)])
        return carry

    lax.fori_loop(0, NGRP, grp_body, 0)


def _gather(cp, idx_flat, tab):
    mesh = plsc.VectorSubcoreMesh(core_axis_name="c", subcore_axis_name="s",
                                  num_cores=NC, num_subcores=NS)
    fn = pl.kernel(
        _gather_body,
        out_type=jax.ShapeDtypeStruct((M * KNN, cp), jnp.float32),
        mesh=mesh,
        scratch_types=[
            pltpu.VMEM((GK // 2,), jnp.int32),
            pltpu.VMEM((GK // 2,), jnp.int32),
            pltpu.VMEM((GK // 2, cp), jnp.float32),
            pltpu.VMEM((GK // 2, cp), jnp.float32),
            pltpu.SemaphoreType.DMA,
            pltpu.SemaphoreType.DMA,
        ],
    )
    return fn(idx_flat, tab)


def _edge_z(c, hg_ref, h_ref, w_ref):
    xi = h_ref[...]
    gg = hg_ref[...][:, :c]
    diff = gg.reshape(BLK_C, KNN, c) - xi[:, None, :]
    xe = jnp.broadcast_to(xi[:, None, :], (BLK_C, KNN, c))
    edge = jnp.concatenate([diff, xe], axis=2).reshape(BLK_C * KNN, 2 * c)
    return jnp.dot(edge.astype(jnp.bfloat16), w_ref[...].astype(jnp.bfloat16),
                   preferred_element_type=jnp.float32)


def _conv1_body(c, hg_ref, h_ref, w_ref, sum_ref):
    i = pl.program_id(0)
    z = _edge_z(c, hg_ref, h_ref, w_ref)

    @pl.when(i == 0)
    def _():
        sum_ref[...] = jnp.zeros_like(sum_ref)

    sum_ref[...] += jnp.sum(z, axis=0, keepdims=True)


def _conv2_body(c, hg_ref, h_ref, w_ref, m_ref, r_ref, sq_ref):
    i = pl.program_id(0)
    z = _edge_z(c, hg_ref, h_ref, w_ref)
    r_ref[...] = jnp.max(z.reshape(BLK_C, KNN, -1), axis=1)

    @pl.when(i == 0)
    def _():
        sq_ref[...] = jnp.zeros_like(sq_ref)

    d = z - m_ref[...]
    sq_ref[...] += jnp.sum(d * d, axis=0, keepdims=True)


def _convz_body(c, hg_ref, h_ref, w_ref, r_ref, z_ref):
    z = _edge_z(c, hg_ref, h_ref, w_ref)
    z_ref[...] = z
    r_ref[...] = jnp.max(z.reshape(BLK_C, KNN, -1), axis=1)


def _conv_zout(hg, h, w2, c, f):
    cp = hg.shape[1]
    const = lambda i: (0, 0)
    return pl.pallas_call(
        functools.partial(_convz_body, c),
        grid=(M // BLK_C,),
        in_specs=[
            pl.BlockSpec((BLK_C * KNN, cp), lambda i: (i, 0)),
            pl.BlockSpec((BLK_C, c), lambda i: (i, 0)),
            pl.BlockSpec((2 * c, f), const),
        ],
        out_specs=[
            pl.BlockSpec((BLK_C, f), lambda i: (i, 0)),
            pl.BlockSpec((BLK_C * KNN, f), lambda i: (i, 0)),
        ],
        out_shape=[
            jax.ShapeDtypeStruct((M, f), jnp.float32),
            jax.ShapeDtypeStruct((M * KNN, f), jnp.float32),
        ],
    )(hg, h, w2)


def _conv(hg, h, w2, c, f):
    cp = hg.shape[1]
    const = lambda i: (0, 0)
    hg_spec = pl.BlockSpec((BLK_C * KNN, cp), lambda i: (i, 0))
    h_spec = pl.BlockSpec((BLK_C, c), lambda i: (i, 0))
    w_spec = pl.BlockSpec((2 * c, f), const)
    s1 = pl.pallas_call(
        functools.partial(_conv1_body, c),
        grid=(M // BLK_C,),
        in_specs=[hg_spec, h_spec, w_spec],
        out_specs=pl.BlockSpec((1, f), const),
        out_shape=jax.ShapeDtypeStruct((1, f), jnp.float32),
    )(hg, h, w2)
    mean = s1 / float(M * KNN)
    r, s2 = pl.pallas_call(
        functools.partial(_conv2_body, c),
        grid=(M // BLK_C,),
        in_specs=[hg_spec, h_spec, w_spec, pl.BlockSpec((1, f), const)],
        out_specs=[
            pl.BlockSpec((BLK_C, f), lambda i: (i, 0)),
            pl.BlockSpec((1, f), const),
        ],
        out_shape=[
            jax.ShapeDtypeStruct((M, f), jnp.float32),
            jax.ShapeDtypeStruct((1, f), jnp.float32),
        ],
    )(hg, h, w2, mean)
    var = s2 / float(M * KNN)
    return r, mean[0], var[0]


def _final_body(h_ref, wf_ref, y_ref, sum_ref):
    i = pl.program_id(0)
    y = jnp.dot(h_ref[...].astype(jnp.bfloat16),
                wf_ref[...].astype(jnp.bfloat16),
                preferred_element_type=jnp.float32)
    y_ref[...] = y

    @pl.when(i == 0)
    def _():
        sum_ref[...] = jnp.zeros_like(sum_ref)

    sum_ref[...] += jnp.sum(y, axis=0, keepdims=True)


def _final_conv(hcat, wf_t, out_dim):
    cdim = hcat.shape[1]
    const = lambda i: (0, 0)
    return pl.pallas_call(
        _final_body,
        grid=(M // BLK_M,),
        in_specs=[
            pl.BlockSpec((BLK_M, cdim), lambda i: (i, 0)),
            pl.BlockSpec((cdim, out_dim), const),
        ],
        out_specs=[
            pl.BlockSpec((BLK_M, out_dim), lambda i: (i, 0)),
            pl.BlockSpec((1, out_dim), const),
        ],
        out_shape=[
            jax.ShapeDtypeStruct((M, out_dim), jnp.float32),
            jax.ShapeDtypeStruct((1, out_dim), jnp.float32),
        ],
    )(hcat, wf_t)


def _var_body(y_ref, m_ref, sq_ref):
    i = pl.program_id(0)

    @pl.when(i == 0)
    def _():
        sq_ref[...] = jnp.zeros_like(sq_ref)

    d = y_ref[...] - m_ref[...]
    sq_ref[...] += jnp.sum(d * d, axis=0, keepdims=True)


def _var_pass(y, mean, out_dim):
    const = lambda i: (0, 0)
    return pl.pallas_call(
        _var_body,
        grid=(M // BLK_M,),
        in_specs=[
            pl.BlockSpec((BLK_M, out_dim), lambda i: (i, 0)),
            pl.BlockSpec((1, out_dim), const),
        ],
        out_specs=pl.BlockSpec((1, out_dim), const),
        out_shape=jax.ShapeDtypeStruct((1, out_dim), jnp.float32),
    )(y, mean)


def _norm_body(m_ref, v_ref, g_ref, b_ref, y_ref, o_ref):
    t = (y_ref[...] - m_ref[...]) / jnp.sqrt(v_ref[...] + EPS)
    t = t * g_ref[...] + b_ref[...]
    o_ref[...] = _leaky(t)


def _normalize(y, m, v, g, b, out_dim):
    const = lambda i: (0, 0)
    return pl.pallas_call(
        _norm_body,
        grid=(M // BLK_M,),
        in_specs=[
            pl.BlockSpec((1, out_dim), const),
            pl.BlockSpec((1, out_dim), const),
            pl.BlockSpec((1, out_dim), const),
            pl.BlockSpec((1, out_dim), const),
            pl.BlockSpec((BLK_M, out_dim), lambda i: (i, 0)),
        ],
        out_specs=pl.BlockSpec((BLK_M, out_dim), lambda i: (i, 0)),
        out_shape=jax.ShapeDtypeStruct((M, out_dim), jnp.float32),
    )(m, v, g, b, y)


def kernel(x, W0, g0, b0, W1, g1, b1, W2, g2, b2, W3, g3, b3, Wf, gf, bf):
    params = [(W0, g0, b0), (W1, g1, b1), (W2, g2, b2), (W3, g3, b3)]
    cnt = float(M * KNN)

    h = x
    feats = []
    for (W, g, bb) in params:
        f, c2 = W.shape
        c = c2 // 2
        ht = jnp.transpose(h, (0, 2, 1))
        xx = jnp.sum(ht ** 2, axis=1, keepdims=True)
        xxt = jnp.transpose(xx, (0, 2, 1))
        idx = _knn(h, ht, xx, xxt)

        cp = max(c, 128)
        tab = h.reshape(M, c)
        if cp != c:
            tab = jnp.pad(tab, ((0, 0), (0, cp - c)))
        hg = _gather(cp, idx.reshape(M * KNN), tab)

        if len(feats) < 3:
            r, z = _conv_zout(hg, h.reshape(M, c), W.T, c, f)
            mean = jnp.mean(z, axis=0)
            z4 = z.reshape(B, N, KNN, f).transpose(0, 3, 1, 2)
            z4 = lax.optimization_barrier(z4)
            var = jnp.mean((z4 - mean.reshape(1, f, 1, 1)) ** 2,
                           axis=(0, 2, 3)).reshape(f)
        else:
            r, mean, var = _conv(hg, h.reshape(M, c), W.T, c, f)
        t = (r - mean) / jnp.sqrt(var + EPS) * g + bb
        hf = jnp.where(t >= 0, t, 0.2 * t)
        feats.append(hf)
        h = hf.reshape(B, N, f)

    out_dim = Wf.shape[0]
    hcat = jnp.concatenate(feats, axis=1)
    y, s1 = _final_conv(hcat, Wf.T, out_dim)
    mean = s1 / float(M)
    var = _var_pass(y, mean, out_dim) / float(M)
    out = _normalize(y, mean, var, gf.reshape(1, out_dim),
                     bf.reshape(1, out_dim), out_dim)
    return out.reshape(B, N, out_dim)
